# Initial kernel scaffold; baseline (speedup 1.0000x reference)
import jax
import jax.numpy as jnp
from jax.experimental import pallas as pl


def kernel(h, W, b, norm, edge_index):
    raise NotImplementedError("write your pallas kernel here")


# speedup vs baseline: 8.0112x; 8.0112x over previous
import functools

import jax
import jax.numpy as jnp
from jax import lax
from jax.experimental import pallas as pl
from jax.experimental.pallas import tpu as pltpu
from jax.experimental.pallas import tpu_sc as plsc

N_NODES = 10000
N_EDGES = 320000
D = 128
NC = 2
NS = 16
L = 16
NW = NC * NS
EPT = N_EDGES // NW
CB = 125
CH = EPT // CB
RPT = N_NODES // NS
ROW_BLK = 1000


def _mm_body(h_ref, n_ref, w_ref, o_ref):
    o_ref[...] = jnp.dot(h_ref[...] * n_ref[...], w_ref[...],
                         preferred_element_type=jnp.float32)


def _fin_body(p_ref, n_ref, b_ref, o_ref):
    s = p_ref[0] + p_ref[1]
    o_ref[...] = jnp.maximum(s * n_ref[...] + b_ref[...], 0.0)


_sc_mesh = plsc.VectorSubcoreMesh(core_axis_name="c", subcore_axis_name="s")


@functools.partial(
    pl.kernel,
    out_type=jax.ShapeDtypeStruct((NC, NS, RPT, D), jnp.float32),
    mesh=_sc_mesh,
    scratch_types=[
        pltpu.VMEM((CH, CB), jnp.int32),
        pltpu.VMEM((CH, CB), jnp.int32),
        pltpu.VMEM((CB, D), jnp.float32),
        pltpu.VMEM_SHARED((N_NODES, D), jnp.float32),
        pltpu.SemaphoreType.DMA,
    ],
)
def _sc_scatter(x_hbm, src_hbm, dst_hbm, out_hbm, src_v, dst_v, rows_v,
                agg_sh, sem):
    cid = lax.axis_index("c")
    sid = lax.axis_index("s")
    wid = cid * NS + sid

    zeros = jnp.zeros((L,), jnp.float32)

    @pl.loop(0, CB)
    def _zero_rows(i):
        for c in range(D // L):
            rows_v[i, pl.ds(c * L, L)] = zeros

    for k in range(RPT // CB):
        pltpu.sync_copy(rows_v, agg_sh.at[pl.ds(sid * RPT + k * CB, CB)])
    plsc.subcore_barrier()

    pltpu.sync_copy(src_hbm.at[wid], src_v)
    pltpu.sync_copy(dst_hbm.at[wid], dst_v)

    @pl.loop(0, CH)
    def _edges(j):
        pltpu.async_copy(x_hbm.at[src_v.at[j]], rows_v, sem).wait()
        pltpu.sync_copy(rows_v, agg_sh.at[dst_v.at[j]], add=True)

    plsc.subcore_barrier()
    pltpu.sync_copy(agg_sh.at[pl.ds(sid * RPT, RPT)], out_hbm.at[cid, sid])


@jax.jit
def kernel(h, W, b, norm, edge_index):
    src = edge_index[0].reshape(NW, CH, CB)
    dst = edge_index[1].reshape(NW, CH, CB)

    x = pl.pallas_call(
        _mm_body,
        grid=(N_NODES // ROW_BLK,),
        in_specs=[
            pl.BlockSpec((ROW_BLK, D), lambda i: (i, 0)),
            pl.BlockSpec((ROW_BLK, 1), lambda i: (i, 0)),
            pl.BlockSpec((D, D), lambda i: (0, 0)),
        ],
        out_specs=pl.BlockSpec((ROW_BLK, D), lambda i: (i, 0)),
        out_shape=jax.ShapeDtypeStruct((N_NODES, D), jnp.float32),
    )(h, norm, W)

    parts = _sc_scatter(x, src, dst).reshape(NC, N_NODES, D)

    out = pl.pallas_call(
        _fin_body,
        grid=(N_NODES // ROW_BLK,),
        in_specs=[
            pl.BlockSpec((NC, ROW_BLK, D), lambda i: (0, i, 0)),
            pl.BlockSpec((ROW_BLK, 1), lambda i: (i, 0)),
            pl.BlockSpec((1, D), lambda i: (0, 0)),
        ],
        out_specs=pl.BlockSpec((ROW_BLK, D), lambda i: (i, 0)),
        out_shape=jax.ShapeDtypeStruct((N_NODES, D), jnp.float32),
    )(parts, norm, b.reshape(1, D))
    return out


# speedup vs baseline: 9.9350x; 1.2401x over previous
import functools

import jax
import jax.numpy as jnp
from jax import lax
from jax.experimental import pallas as pl
from jax.experimental.pallas import tpu as pltpu
from jax.experimental.pallas import tpu_sc as plsc

N_NODES = 10000
N_EDGES = 320000
D = 128
NC = 2
NS = 16
L = 16
NW = NC * NS
EPT = N_EDGES // NW
CB = 125
CH = EPT // CB
RPT = N_NODES // NS
ROW_BLK = 1000


def _mm_body(h_ref, n_ref, w_ref, o_ref):
    o_ref[...] = jnp.dot(h_ref[...] * n_ref[...], w_ref[...],
                         preferred_element_type=jnp.float32)


def _fin_body(p_ref, n_ref, b_ref, o_ref):
    s = p_ref[0] + p_ref[1]
    o_ref[...] = jnp.maximum(s * n_ref[...] + b_ref[...], 0.0)


_sc_mesh = plsc.VectorSubcoreMesh(core_axis_name="c", subcore_axis_name="s")


@functools.partial(
    pl.kernel,
    out_type=jax.ShapeDtypeStruct((NC, NS, RPT, D), jnp.float32),
    mesh=_sc_mesh,
    scratch_types=[
        pltpu.VMEM((2, CB), jnp.int32),
        pltpu.VMEM((2, CB), jnp.int32),
        pltpu.VMEM((CB, D), jnp.float32),
        pltpu.VMEM((CB, D), jnp.float32),
        pltpu.VMEM_SHARED((N_NODES, D), jnp.float32),
        pltpu.SemaphoreType.DMA,
        pltpu.SemaphoreType.DMA,
        pltpu.SemaphoreType.DMA,
        pltpu.SemaphoreType.DMA,
    ],
)
def _sc_scatter(x_hbm, src_hbm, dst_hbm, out_hbm, sbuf, dbuf, rows_a,
                rows_b, agg_sh, gsem_a, gsem_b, isem_a, isem_b):
    cid = lax.axis_index("c")
    sid = lax.axis_index("s")
    wid = cid * NS + sid
    rows = (rows_a, rows_b)
    gsems = (gsem_a, gsem_b)
    isems = (isem_a, isem_b)

    zeros = jnp.zeros((L,), jnp.float32)

    @pl.loop(0, CB)
    def _zero_rows(i):
        for c in range(D // L):
            rows_a[i, pl.ds(c * L, L)] = zeros

    for k in range(RPT // CB):
        pltpu.sync_copy(rows_a, agg_sh.at[pl.ds(sid * RPT + k * CB, CB)])
    plsc.subcore_barrier()

    def start_idx(j, p):
        pltpu.async_copy(src_hbm.at[wid, j], sbuf.at[p], isems[p])
        pltpu.async_copy(dst_hbm.at[wid, j], dbuf.at[p], isems[p])

    def wait_idx(j, p):
        pltpu.make_async_copy(src_hbm.at[wid, j], sbuf.at[p],
                              isems[p]).wait()
        pltpu.make_async_copy(dst_hbm.at[wid, j], dbuf.at[p],
                              isems[p]).wait()

    def start_gather(p):
        pltpu.async_copy(x_hbm.at[sbuf.at[p]], rows[p], gsems[p])

    def wait_gather(p):
        pltpu.make_async_copy(x_hbm.at[sbuf.at[p]], rows[p],
                              gsems[p]).wait()

    start_idx(0, 0)
    wait_idx(0, 0)
    start_gather(0)
    start_idx(1, 1)

    @pl.loop(0, CH, step=2)
    def _edges(j):
        for p in range(2):
            cj = j + p
            wait_gather(p)

            @pl.when(cj + 1 < CH)
            def _next_gather():
                wait_idx(cj + 1, 1 - p)
                start_gather(1 - p)

            pltpu.sync_copy(rows[p], agg_sh.at[dbuf.at[p]], add=True)

            @pl.when(cj + 2 < CH)
            def _next_idx():
                start_idx(cj + 2, p)

    plsc.subcore_barrier()
    pltpu.sync_copy(agg_sh.at[pl.ds(sid * RPT, RPT)], out_hbm.at[cid, sid])


@jax.jit
def kernel(h, W, b, norm, edge_index):
    src = edge_index[0].reshape(NW, CH, CB)
    dst = edge_index[1].reshape(NW, CH, CB)

    x = pl.pallas_call(
        _mm_body,
        grid=(N_NODES // ROW_BLK,),
        in_specs=[
            pl.BlockSpec((ROW_BLK, D), lambda i: (i, 0)),
            pl.BlockSpec((ROW_BLK, 1), lambda i: (i, 0)),
            pl.BlockSpec((D, D), lambda i: (0, 0)),
        ],
        out_specs=pl.BlockSpec((ROW_BLK, D), lambda i: (i, 0)),
        out_shape=jax.ShapeDtypeStruct((N_NODES, D), jnp.float32),
    )(h, norm, W)

    parts = _sc_scatter(x, src, dst).reshape(NC, N_NODES, D)

    out = pl.pallas_call(
        _fin_body,
        grid=(N_NODES // ROW_BLK,),
        in_specs=[
            pl.BlockSpec((NC, ROW_BLK, D), lambda i: (0, i, 0)),
            pl.BlockSpec((ROW_BLK, 1), lambda i: (i, 0)),
            pl.BlockSpec((1, D), lambda i: (0, 0)),
        ],
        out_specs=pl.BlockSpec((ROW_BLK, D), lambda i: (i, 0)),
        out_shape=jax.ShapeDtypeStruct((N_NODES, D), jnp.float32),
    )(parts, norm, b.reshape(1, D))
    return out


# speedup vs baseline: 11.0748x; 1.1147x over previous
import functools

import jax
import jax.numpy as jnp
from jax import lax
from jax.experimental import pallas as pl
from jax.experimental.pallas import tpu as pltpu
from jax.experimental.pallas import tpu_sc as plsc

N_NODES = 10000
N_EDGES = 320000
D = 128
NC = 2
NS = 16
L = 16
NW = NC * NS
EPT = N_EDGES // NW
CB = 125
CH = EPT // CB
RPT = N_NODES // NS
ROW_BLK = 1000


def _mm_body(h_ref, n_ref, w_ref, o_ref):
    o_ref[...] = jnp.dot(h_ref[...] * n_ref[...], w_ref[...],
                         preferred_element_type=jnp.float32)


def _fin_body(p_ref, n_ref, b_ref, o_ref):
    s = p_ref[0] + p_ref[1]
    o_ref[...] = jnp.maximum(s * n_ref[...] + b_ref[...], 0.0)


_sc_mesh = plsc.VectorSubcoreMesh(core_axis_name="c", subcore_axis_name="s")


@functools.partial(
    pl.kernel,
    out_type=jax.ShapeDtypeStruct((NC, N_NODES, D), jnp.float32),
    mesh=_sc_mesh,
    scratch_types=[
        pltpu.VMEM((2, CB), jnp.int32),
        pltpu.VMEM((2, CB), jnp.int32),
        pltpu.VMEM((CB, D), jnp.float32),
        pltpu.VMEM((CB, D), jnp.float32),
        pltpu.VMEM_SHARED((N_NODES, D), jnp.float32),
        pltpu.SemaphoreType.DMA,
        pltpu.SemaphoreType.DMA,
        pltpu.SemaphoreType.DMA,
        pltpu.SemaphoreType.DMA,
    ],
)
def _sc_scatter(x_hbm, ei_hbm, out_hbm, sbuf, dbuf, rows_a,
                rows_b, agg_sh, gsem_a, gsem_b, isem_a, isem_b):
    cid = lax.axis_index("c")
    sid = lax.axis_index("s")
    wid = cid * NS + sid
    rows = (rows_a, rows_b)
    gsems = (gsem_a, gsem_b)
    isems = (isem_a, isem_b)

    zeros = jnp.zeros((L,), jnp.float32)

    @pl.loop(0, CB)
    def _zero_rows(i):
        for c in range(D // L):
            rows_a[i, pl.ds(c * L, L)] = zeros

    for k in range(RPT // CB):
        pltpu.sync_copy(rows_a, agg_sh.at[pl.ds(sid * RPT + k * CB, CB)])
    plsc.subcore_barrier()

    def start_idx(j, p):
        pltpu.async_copy(ei_hbm.at[0, wid, j], sbuf.at[p], isems[p])
        pltpu.async_copy(ei_hbm.at[1, wid, j], dbuf.at[p], isems[p])

    def wait_idx(j, p):
        pltpu.make_async_copy(ei_hbm.at[0, wid, j], sbuf.at[p],
                              isems[p]).wait()
        pltpu.make_async_copy(ei_hbm.at[1, wid, j], dbuf.at[p],
                              isems[p]).wait()

    def start_gather(p):
        pltpu.async_copy(x_hbm.at[sbuf.at[p]], rows[p], gsems[p])

    def wait_gather(p):
        pltpu.make_async_copy(x_hbm.at[sbuf.at[p]], rows[p],
                              gsems[p]).wait()

    start_idx(0, 0)
    wait_idx(0, 0)
    start_gather(0)
    start_idx(1, 1)

    @pl.loop(0, CH, step=2)
    def _edges(j):
        for p in range(2):
            cj = j + p
            wait_gather(p)

            @pl.when(cj + 1 < CH)
            def _next_gather():
                wait_idx(cj + 1, 1 - p)
                start_gather(1 - p)

            pltpu.sync_copy(rows[p], agg_sh.at[dbuf.at[p]], add=True)

            @pl.when(cj + 2 < CH)
            def _next_idx():
                start_idx(cj + 2, p)

    plsc.subcore_barrier()

    @pl.when(sid < NS - 1)
    def _copy_out():
        pltpu.sync_copy(agg_sh.at[pl.ds(sid * 624, 624)],
                        out_hbm.at[cid, pl.ds(sid * 624, 624)])

    @pl.when(sid == NS - 1)
    def _copy_out_last():
        pltpu.sync_copy(agg_sh.at[pl.ds(9360, 640)],
                        out_hbm.at[cid, pl.ds(9360, 640)])


@jax.jit
def kernel(h, W, b, norm, edge_index):
    ei = edge_index.reshape(2, NW, CH, CB)

    x = pl.pallas_call(
        _mm_body,
        grid=(N_NODES // ROW_BLK,),
        in_specs=[
            pl.BlockSpec((ROW_BLK, D), lambda i: (i, 0)),
            pl.BlockSpec((ROW_BLK, 1), lambda i: (i, 0)),
            pl.BlockSpec((D, D), lambda i: (0, 0)),
        ],
        out_specs=pl.BlockSpec((ROW_BLK, D), lambda i: (i, 0)),
        out_shape=jax.ShapeDtypeStruct((N_NODES, D), jnp.float32),
    )(h, norm, W)

    parts = _sc_scatter(x, ei)

    out = pl.pallas_call(
        _fin_body,
        grid=(N_NODES // ROW_BLK,),
        in_specs=[
            pl.BlockSpec((NC, ROW_BLK, D), lambda i: (0, i, 0)),
            pl.BlockSpec((ROW_BLK, 1), lambda i: (i, 0)),
            pl.BlockSpec((1, D), lambda i: (0, 0)),
        ],
        out_specs=pl.BlockSpec((ROW_BLK, D), lambda i: (i, 0)),
        out_shape=jax.ShapeDtypeStruct((N_NODES, D), jnp.float32),
    )(parts, norm, b.reshape(1, D))
    return out


# speedup vs baseline: 11.6560x; 1.0525x over previous
import functools

import jax
import jax.numpy as jnp
from jax import lax
from jax.experimental import pallas as pl
from jax.experimental.pallas import tpu as pltpu
from jax.experimental.pallas import tpu_sc as plsc

N_NODES = 10000
N_EDGES = 320000
D = 128
NC = 2
NS = 16
L = 16
NW = NC * NS
CB = 128
NCHUNK = N_EDGES // CB
TRIPS = -(-NCHUNK // NW)
RPT = N_NODES // NS
ROW_BLK = 1000


def _mm_body(h_ref, n_ref, w_ref, o_ref):
    o_ref[...] = jnp.dot(h_ref[...] * n_ref[...], w_ref[...],
                         preferred_element_type=jnp.float32)


def _fin_body(p_ref, n_ref, b_ref, o_ref):
    s = p_ref[0] + p_ref[1]
    o_ref[...] = jnp.maximum(s * n_ref[...] + b_ref[...], 0.0)


_sc_mesh = plsc.VectorSubcoreMesh(core_axis_name="c", subcore_axis_name="s")


@functools.partial(
    pl.kernel,
    out_type=jax.ShapeDtypeStruct((NC, N_NODES, D), jnp.float32),
    mesh=_sc_mesh,
    scratch_types=[
        pltpu.VMEM((2, 2, CB), jnp.int32),
        pltpu.VMEM((CB, D), jnp.float32),
        pltpu.VMEM((CB, D), jnp.float32),
        pltpu.VMEM_SHARED((N_NODES, D), jnp.float32),
        pltpu.SemaphoreType.DMA,
        pltpu.SemaphoreType.DMA,
        pltpu.SemaphoreType.DMA,
        pltpu.SemaphoreType.DMA,
    ],
)
def _sc_scatter(x_hbm, ei_hbm, out_hbm, ibuf, rows_a, rows_b, agg_sh,
                gsem_a, gsem_b, isem_a, isem_b):
    cid = lax.axis_index("c")
    sid = lax.axis_index("s")
    wid = cid * NS + sid
    rows = (rows_a, rows_b)
    gsems = (gsem_a, gsem_b)
    isems = (isem_a, isem_b)

    zeros = jnp.zeros((L,), jnp.float32)

    @pl.loop(0, CB)
    def _zero_rows(i):
        for c in range(D // L):
            rows_a[i, pl.ds(c * L, L)] = zeros

    for k in range(RPT // CB):
        pltpu.sync_copy(rows_a, agg_sh.at[pl.ds(sid * RPT + k * CB, CB)])
    if RPT % CB:
        pltpu.sync_copy(
            rows_a.at[pl.ds(0, RPT % CB)],
            agg_sh.at[pl.ds(sid * RPT + (RPT // CB) * CB, RPT % CB)])
    plsc.subcore_barrier()

    def start_idx(c, p):
        pltpu.async_copy(ei_hbm.at[:, pl.ds(c * CB, CB)], ibuf.at[p],
                         isems[p])

    def wait_idx(c, p):
        pltpu.make_async_copy(ei_hbm.at[:, pl.ds(c * CB, CB)], ibuf.at[p],
                              isems[p]).wait()

    def start_gather(p):
        pltpu.async_copy(x_hbm.at[ibuf.at[p, 0]], rows[p], gsems[p])

    def wait_gather(p):
        pltpu.make_async_copy(x_hbm.at[ibuf.at[p, 0]], rows[p],
                              gsems[p]).wait()

    start_idx(wid, 0)
    wait_idx(wid, 0)
    start_gather(0)

    @pl.when(wid + NW < NCHUNK)
    def _prime_idx():
        start_idx(wid + NW, 1)

    @pl.loop(0, TRIPS, step=2)
    def _edges(i):
        for p in range(2):
            c = wid + (i + p) * NW

            @pl.when(c < NCHUNK)
            def _chunk():
                wait_gather(p)

                @pl.when(c + NW < NCHUNK)
                def _next_gather():
                    wait_idx(c + NW, 1 - p)
                    start_gather(1 - p)

                pltpu.sync_copy(rows[p], agg_sh.at[ibuf.at[p, 1]], add=True)

                @pl.when(c + 2 * NW < NCHUNK)
                def _next_idx():
                    start_idx(c + 2 * NW, p)

    plsc.subcore_barrier()

    @pl.when(sid < NS - 1)
    def _copy_out():
        pltpu.sync_copy(agg_sh.at[pl.ds(sid * 624, 624)],
                        out_hbm.at[cid, pl.ds(sid * 624, 624)])

    @pl.when(sid == NS - 1)
    def _copy_out_last():
        pltpu.sync_copy(agg_sh.at[pl.ds(9360, 640)],
                        out_hbm.at[cid, pl.ds(9360, 640)])


@jax.jit
def kernel(h, W, b, norm, edge_index):
    x = pl.pallas_call(
        _mm_body,
        grid=(N_NODES // ROW_BLK,),
        in_specs=[
            pl.BlockSpec((ROW_BLK, D), lambda i: (i, 0)),
            pl.BlockSpec((ROW_BLK, 1), lambda i: (i, 0)),
            pl.BlockSpec((D, D), lambda i: (0, 0)),
        ],
        out_specs=pl.BlockSpec((ROW_BLK, D), lambda i: (i, 0)),
        out_shape=jax.ShapeDtypeStruct((N_NODES, D), jnp.float32),
    )(h, norm, W)

    parts = _sc_scatter(x, edge_index)

    out = pl.pallas_call(
        _fin_body,
        grid=(N_NODES // ROW_BLK,),
        in_specs=[
            pl.BlockSpec((NC, ROW_BLK, D), lambda i: (0, i, 0)),
            pl.BlockSpec((ROW_BLK, 1), lambda i: (i, 0)),
            pl.BlockSpec((1, D), lambda i: (0, 0)),
        ],
        out_specs=pl.BlockSpec((ROW_BLK, D), lambda i: (i, 0)),
        out_shape=jax.ShapeDtypeStruct((N_NODES, D), jnp.float32),
    )(parts, norm, b.reshape(1, D))
    return out


# speedup vs baseline: 11.9183x; 1.0225x over previous
import functools

import jax
import jax.numpy as jnp
from jax import lax
from jax.experimental import pallas as pl
from jax.experimental.pallas import tpu as pltpu
from jax.experimental.pallas import tpu_sc as plsc

N_NODES = 10000
N_EDGES = 320000
D = 128
NC = 2
NS = 16
L = 16
NW = NC * NS
CB = 128
NCHUNK = N_EDGES // CB
TRIPS = -(-NCHUNK // NW)
RPT = N_NODES // NS
ROW_BLK = 1000


def _mm_body(h_ref, n_ref, w_ref, o_ref):
    o_ref[...] = jnp.dot(h_ref[...] * n_ref[...], w_ref[...],
                         preferred_element_type=jnp.float32)


def _fin_body(p_ref, n_ref, b_ref, o_ref):
    s = p_ref[0] + p_ref[1]
    o_ref[...] = jnp.maximum(s * n_ref[...] + b_ref[...], 0.0)


_sc_mesh = plsc.VectorSubcoreMesh(core_axis_name="c", subcore_axis_name="s")


@functools.partial(
    pl.kernel,
    out_type=jax.ShapeDtypeStruct((NC, N_NODES, D), jnp.float32),
    mesh=_sc_mesh,
    scratch_types=[
        pltpu.VMEM((2, 2, CB), jnp.int32),
        pltpu.VMEM((CB, D), jnp.float32),
        pltpu.VMEM((CB, D), jnp.float32),
        pltpu.VMEM_SHARED((N_NODES, D), jnp.float32),
        pltpu.SemaphoreType.DMA,
        pltpu.SemaphoreType.DMA,
        pltpu.SemaphoreType.DMA,
        pltpu.SemaphoreType.DMA,
        pltpu.SemaphoreType.DMA,
        pltpu.SemaphoreType.DMA,
    ],
)
def _sc_scatter(x_hbm, ei_hbm, out_hbm, ibuf, rows_a, rows_b, agg_sh,
                gsem_a, gsem_b, gsem_c, gsem_d, isem_a, isem_b):
    cid = lax.axis_index("c")
    sid = lax.axis_index("s")
    wid = cid * NS + sid
    rows = (rows_a, rows_b)
    gsems = ((gsem_a, gsem_c), (gsem_b, gsem_d))
    isems = (isem_a, isem_b)

    zeros = jnp.zeros((L,), jnp.float32)

    @pl.loop(0, CB)
    def _zero_rows(i):
        for c in range(D // L):
            rows_a[i, pl.ds(c * L, L)] = zeros

    for k in range(RPT // CB):
        pltpu.sync_copy(rows_a, agg_sh.at[pl.ds(sid * RPT + k * CB, CB)])
    if RPT % CB:
        pltpu.sync_copy(
            rows_a.at[pl.ds(0, RPT % CB)],
            agg_sh.at[pl.ds(sid * RPT + (RPT // CB) * CB, RPT % CB)])
    plsc.subcore_barrier()

    def start_idx(c, p):
        pltpu.async_copy(ei_hbm.at[:, pl.ds(c * CB, CB)], ibuf.at[p],
                         isems[p])

    def wait_idx(c, p):
        pltpu.make_async_copy(ei_hbm.at[:, pl.ds(c * CB, CB)], ibuf.at[p],
                              isems[p]).wait()

    H = CB // 2

    def start_gather(p):
        pltpu.async_copy(x_hbm.at[ibuf.at[p, 0, pl.ds(0, H)]],
                         rows[p].at[pl.ds(0, H)], gsems[p][0])
        pltpu.async_copy(x_hbm.at[ibuf.at[p, 0, pl.ds(H, H)]],
                         rows[p].at[pl.ds(H, H)], gsems[p][1])

    def wait_gather(p):
        pltpu.make_async_copy(x_hbm.at[ibuf.at[p, 0, pl.ds(0, H)]],
                              rows[p].at[pl.ds(0, H)], gsems[p][0]).wait()
        pltpu.make_async_copy(x_hbm.at[ibuf.at[p, 0, pl.ds(H, H)]],
                              rows[p].at[pl.ds(H, H)], gsems[p][1]).wait()

    start_idx(wid, 0)
    wait_idx(wid, 0)
    start_gather(0)

    @pl.when(wid + NW < NCHUNK)
    def _prime_idx():
        start_idx(wid + NW, 1)

    @pl.loop(0, TRIPS, step=2)
    def _edges(i):
        for p in range(2):
            c = wid + (i + p) * NW

            @pl.when(c < NCHUNK)
            def _chunk():
                wait_gather(p)

                @pl.when(c + NW < NCHUNK)
                def _next_gather():
                    wait_idx(c + NW, 1 - p)
                    start_gather(1 - p)

                pltpu.sync_copy(rows[p], agg_sh.at[ibuf.at[p, 1]], add=True)

                @pl.when(c + 2 * NW < NCHUNK)
                def _next_idx():
                    start_idx(c + 2 * NW, p)

    plsc.subcore_barrier()

    @pl.when(sid < NS - 1)
    def _copy_out():
        pltpu.sync_copy(agg_sh.at[pl.ds(sid * 624, 624)],
                        out_hbm.at[cid, pl.ds(sid * 624, 624)])

    @pl.when(sid == NS - 1)
    def _copy_out_last():
        pltpu.sync_copy(agg_sh.at[pl.ds(9360, 640)],
                        out_hbm.at[cid, pl.ds(9360, 640)])


@jax.jit
def kernel(h, W, b, norm, edge_index):
    x = pl.pallas_call(
        _mm_body,
        grid=(N_NODES // ROW_BLK,),
        in_specs=[
            pl.BlockSpec((ROW_BLK, D), lambda i: (i, 0)),
            pl.BlockSpec((ROW_BLK, 1), lambda i: (i, 0)),
            pl.BlockSpec((D, D), lambda i: (0, 0)),
        ],
        out_specs=pl.BlockSpec((ROW_BLK, D), lambda i: (i, 0)),
        out_shape=jax.ShapeDtypeStruct((N_NODES, D), jnp.float32),
    )(h, norm, W)

    parts = _sc_scatter(x, edge_index)

    out = pl.pallas_call(
        _fin_body,
        grid=(N_NODES // ROW_BLK,),
        in_specs=[
            pl.BlockSpec((NC, ROW_BLK, D), lambda i: (0, i, 0)),
            pl.BlockSpec((ROW_BLK, 1), lambda i: (i, 0)),
            pl.BlockSpec((1, D), lambda i: (0, 0)),
        ],
        out_specs=pl.BlockSpec((ROW_BLK, D), lambda i: (i, 0)),
        out_shape=jax.ShapeDtypeStruct((N_NODES, D), jnp.float32),
    )(parts, norm, b.reshape(1, D))
    return out


# speedup vs baseline: 12.2522x; 1.0280x over previous
import functools

import jax
import jax.numpy as jnp
from jax import lax
from jax.experimental import pallas as pl
from jax.experimental.pallas import tpu as pltpu
from jax.experimental.pallas import tpu_sc as plsc

N_NODES = 10000
N_EDGES = 320000
D = 128
NC = 2
NS = 16
L = 16
NW = NC * NS
CB = 128
NCHUNK = N_EDGES // CB
TRIPS = -(-NCHUNK // NW)
RPT = N_NODES // NS
ROW_BLK = 2000


def _mm_body(h_ref, n_ref, w_ref, o_ref):
    o_ref[...] = jnp.dot(h_ref[...] * n_ref[...], w_ref[...],
                         preferred_element_type=jnp.float32)


def _fin_body(p_ref, n_ref, b_ref, o_ref):
    s = p_ref[0] + p_ref[1]
    o_ref[...] = jnp.maximum(s * n_ref[...] + b_ref[...], 0.0)


_sc_mesh = plsc.VectorSubcoreMesh(core_axis_name="c", subcore_axis_name="s")


@functools.partial(
    pl.kernel,
    out_type=jax.ShapeDtypeStruct((NC, N_NODES, D), jnp.float32),
    mesh=_sc_mesh,
    scratch_types=[
        pltpu.VMEM((2, 2, CB), jnp.int32),
        pltpu.VMEM((CB, D), jnp.float32),
        pltpu.VMEM((CB, D), jnp.float32),
        pltpu.VMEM_SHARED((N_NODES, D), jnp.float32),
        pltpu.SemaphoreType.DMA,
        pltpu.SemaphoreType.DMA,
        pltpu.SemaphoreType.DMA,
        pltpu.SemaphoreType.DMA,
        pltpu.SemaphoreType.DMA,
        pltpu.SemaphoreType.DMA,
    ],
)
def _sc_scatter(x_hbm, ei_hbm, out_hbm, ibuf, rows_a, rows_b, agg_sh,
                gsem_a, gsem_b, gsem_c, gsem_d, isem_a, isem_b):
    cid = lax.axis_index("c")
    sid = lax.axis_index("s")
    wid = cid * NS + sid
    rows = (rows_a, rows_b)
    gsems = ((gsem_a, gsem_c), (gsem_b, gsem_d))
    isems = (isem_a, isem_b)

    zeros = jnp.zeros((L,), jnp.float32)

    @pl.loop(0, CB)
    def _zero_rows(i):
        for c in range(D // L):
            rows_a[i, pl.ds(c * L, L)] = zeros

    for k in range(RPT // CB):
        pltpu.sync_copy(rows_a, agg_sh.at[pl.ds(sid * RPT + k * CB, CB)])
    if RPT % CB:
        pltpu.sync_copy(
            rows_a.at[pl.ds(0, RPT % CB)],
            agg_sh.at[pl.ds(sid * RPT + (RPT // CB) * CB, RPT % CB)])
    plsc.subcore_barrier()

    def start_idx(c, p):
        pltpu.async_copy(ei_hbm.at[:, pl.ds(c * CB, CB)], ibuf.at[p],
                         isems[p])

    def wait_idx(c, p):
        pltpu.make_async_copy(ei_hbm.at[:, pl.ds(c * CB, CB)], ibuf.at[p],
                              isems[p]).wait()

    H = CB // 2

    def start_gather(p):
        pltpu.async_copy(x_hbm.at[ibuf.at[p, 0, pl.ds(0, H)]],
                         rows[p].at[pl.ds(0, H)], gsems[p][0])
        pltpu.async_copy(x_hbm.at[ibuf.at[p, 0, pl.ds(H, H)]],
                         rows[p].at[pl.ds(H, H)], gsems[p][1])

    def wait_gather(p):
        pltpu.make_async_copy(x_hbm.at[ibuf.at[p, 0, pl.ds(0, H)]],
                              rows[p].at[pl.ds(0, H)], gsems[p][0]).wait()
        pltpu.make_async_copy(x_hbm.at[ibuf.at[p, 0, pl.ds(H, H)]],
                              rows[p].at[pl.ds(H, H)], gsems[p][1]).wait()

    start_idx(wid, 0)
    wait_idx(wid, 0)
    start_gather(0)

    @pl.when(wid + NW < NCHUNK)
    def _prime_idx():
        start_idx(wid + NW, 1)

    @pl.loop(0, TRIPS, step=2)
    def _edges(i):
        for p in range(2):
            c = wid + (i + p) * NW

            @pl.when(c < NCHUNK)
            def _chunk():
                wait_gather(p)

                @pl.when(c + NW < NCHUNK)
                def _next_gather():
                    wait_idx(c + NW, 1 - p)
                    start_gather(1 - p)

                pltpu.sync_copy(rows[p], agg_sh.at[ibuf.at[p, 1]], add=True)

                @pl.when(c + 2 * NW < NCHUNK)
                def _next_idx():
                    start_idx(c + 2 * NW, p)

    plsc.subcore_barrier()

    @pl.when(sid < NS - 1)
    def _copy_out():
        pltpu.sync_copy(agg_sh.at[pl.ds(sid * 624, 624)],
                        out_hbm.at[cid, pl.ds(sid * 624, 624)])

    @pl.when(sid == NS - 1)
    def _copy_out_last():
        pltpu.sync_copy(agg_sh.at[pl.ds(9360, 640)],
                        out_hbm.at[cid, pl.ds(9360, 640)])


@jax.jit
def kernel(h, W, b, norm, edge_index):
    x = pl.pallas_call(
        _mm_body,
        grid=(N_NODES // ROW_BLK,),
        in_specs=[
            pl.BlockSpec((ROW_BLK, D), lambda i: (i, 0)),
            pl.BlockSpec((ROW_BLK, 1), lambda i: (i, 0)),
            pl.BlockSpec((D, D), lambda i: (0, 0)),
        ],
        out_specs=pl.BlockSpec((ROW_BLK, D), lambda i: (i, 0)),
        out_shape=jax.ShapeDtypeStruct((N_NODES, D), jnp.float32),
    )(h, norm, W)

    parts = _sc_scatter(x, edge_index)

    out = pl.pallas_call(
        _fin_body,
        grid=(N_NODES // ROW_BLK,),
        in_specs=[
            pl.BlockSpec((NC, ROW_BLK, D), lambda i: (0, i, 0)),
            pl.BlockSpec((ROW_BLK, 1), lambda i: (i, 0)),
            pl.BlockSpec((1, D), lambda i: (0, 0)),
        ],
        out_specs=pl.BlockSpec((ROW_BLK, D), lambda i: (i, 0)),
        out_shape=jax.ShapeDtypeStruct((N_NODES, D), jnp.float32),
    )(parts, norm, b.reshape(1, D))
    return out


# speedup vs baseline: 12.5643x; 1.0255x over previous
import functools

import jax
import jax.numpy as jnp
from jax import lax
from jax.experimental import pallas as pl
from jax.experimental.pallas import tpu as pltpu
from jax.experimental.pallas import tpu_sc as plsc

N_NODES = 10000
N_EDGES = 320000
D = 128
NC = 2
NS = 16
L = 16
NW = NC * NS
CB = 128
NCHUNK = N_EDGES // CB
TRIPS = -(-NCHUNK // NW)
RPT = N_NODES // NS
ROW_BLK = 2000


def _mm_body(h_ref, n_ref, w_ref, o_ref):
    o_ref[...] = jnp.dot(h_ref[...] * n_ref[...], w_ref[...],
                         preferred_element_type=jnp.float32)


def _fin_body(p_ref, n_ref, b_ref, o_ref):
    s = p_ref[0] + p_ref[1]
    o_ref[...] = jnp.maximum(s * n_ref[...] + b_ref[...], 0.0)


_sc_mesh = plsc.VectorSubcoreMesh(core_axis_name="c", subcore_axis_name="s")


@functools.partial(
    pl.kernel,
    out_type=jax.ShapeDtypeStruct((NC, N_NODES, D), jnp.float32),
    mesh=_sc_mesh,
    scratch_types=[
        pltpu.VMEM((2, 2, CB), jnp.int32),
        pltpu.VMEM((CB, D), jnp.float32),
        pltpu.VMEM((CB, D), jnp.float32),
        pltpu.VMEM_SHARED((N_NODES, D), jnp.float32),
        pltpu.SemaphoreType.DMA,
        pltpu.SemaphoreType.DMA,
        pltpu.SemaphoreType.DMA,
        pltpu.SemaphoreType.DMA,
        pltpu.SemaphoreType.DMA,
        pltpu.SemaphoreType.DMA,
    ],
)
def _sc_scatter(x_hbm, ei_hbm, out_hbm, ibuf, rows_a, rows_b, agg_sh,
                gsem_a, gsem_b, gsem_c, gsem_d, isem_a, isem_b):
    cid = lax.axis_index("c")
    sid = lax.axis_index("s")
    wid = cid * NS + sid
    rows = (rows_a, rows_b)
    gsems = ((gsem_a, gsem_c), (gsem_b, gsem_d))
    isems = (isem_a, isem_b)

    zeros = jnp.zeros((L,), jnp.float32)

    @pl.loop(0, CB)
    def _zero_rows(i):
        for c in range(D // L):
            rows_a[i, pl.ds(c * L, L)] = zeros

    for k in range(RPT // CB):
        pltpu.sync_copy(rows_a, agg_sh.at[pl.ds(sid * RPT + k * CB, CB)])
    if RPT % CB:
        pltpu.sync_copy(
            rows_a.at[pl.ds(0, RPT % CB)],
            agg_sh.at[pl.ds(sid * RPT + (RPT // CB) * CB, RPT % CB)])
    plsc.subcore_barrier()

    def start_idx(c, p):
        pltpu.async_copy(ei_hbm.at[:, pl.ds(c * CB, CB)], ibuf.at[p],
                         isems[p])

    def wait_idx(c, p):
        pltpu.make_async_copy(ei_hbm.at[:, pl.ds(c * CB, CB)], ibuf.at[p],
                              isems[p]).wait()

    H = CB // 2

    def start_gather(p):
        pltpu.async_copy(x_hbm.at[ibuf.at[p, 0, pl.ds(0, H)]],
                         rows[p].at[pl.ds(0, H)], gsems[p][0])
        pltpu.async_copy(x_hbm.at[ibuf.at[p, 0, pl.ds(H, H)]],
                         rows[p].at[pl.ds(H, H)], gsems[p][1])

    def wait_gather(p):
        pltpu.make_async_copy(x_hbm.at[ibuf.at[p, 0, pl.ds(0, H)]],
                              rows[p].at[pl.ds(0, H)], gsems[p][0]).wait()
        pltpu.make_async_copy(x_hbm.at[ibuf.at[p, 0, pl.ds(H, H)]],
                              rows[p].at[pl.ds(H, H)], gsems[p][1]).wait()

    start_idx(wid, 0)
    wait_idx(wid, 0)
    start_gather(0)

    @pl.when(wid + NW < NCHUNK)
    def _prime_idx():
        start_idx(wid + NW, 1)

    @pl.loop(0, TRIPS, step=2)
    def _edges(i):
        for p in range(2):
            c = wid + (i + p) * NW

            @pl.when(c < NCHUNK)
            def _chunk():
                @pl.when(c + NW < NCHUNK)
                def _next_gather():
                    wait_idx(c + NW, 1 - p)
                    start_gather(1 - p)

                wait_gather(p)
                pltpu.sync_copy(rows[p], agg_sh.at[ibuf.at[p, 1]], add=True)

                @pl.when(c + 2 * NW < NCHUNK)
                def _next_idx():
                    start_idx(c + 2 * NW, p)

    plsc.subcore_barrier()

    @pl.when(sid < NS - 1)
    def _copy_out():
        pltpu.sync_copy(agg_sh.at[pl.ds(sid * 624, 624)],
                        out_hbm.at[cid, pl.ds(sid * 624, 624)])

    @pl.when(sid == NS - 1)
    def _copy_out_last():
        pltpu.sync_copy(agg_sh.at[pl.ds(9360, 640)],
                        out_hbm.at[cid, pl.ds(9360, 640)])


@jax.jit
def kernel(h, W, b, norm, edge_index):
    x = pl.pallas_call(
        _mm_body,
        grid=(N_NODES // ROW_BLK,),
        in_specs=[
            pl.BlockSpec((ROW_BLK, D), lambda i: (i, 0)),
            pl.BlockSpec((ROW_BLK, 1), lambda i: (i, 0)),
            pl.BlockSpec((D, D), lambda i: (0, 0)),
        ],
        out_specs=pl.BlockSpec((ROW_BLK, D), lambda i: (i, 0)),
        out_shape=jax.ShapeDtypeStruct((N_NODES, D), jnp.float32),
    )(h, norm, W)

    parts = _sc_scatter(x, edge_index)

    out = pl.pallas_call(
        _fin_body,
        grid=(N_NODES // ROW_BLK,),
        in_specs=[
            pl.BlockSpec((NC, ROW_BLK, D), lambda i: (0, i, 0)),
            pl.BlockSpec((ROW_BLK, 1), lambda i: (i, 0)),
            pl.BlockSpec((1, D), lambda i: (0, 0)),
        ],
        out_specs=pl.BlockSpec((ROW_BLK, D), lambda i: (i, 0)),
        out_shape=jax.ShapeDtypeStruct((N_NODES, D), jnp.float32),
    )(parts, norm, b.reshape(1, D))
    return out
